# baseline (speedup 1.0000x reference)
import jax
import jax.numpy as jnp
import numpy as np
from functools import partial
from jax.experimental import pallas as pl
from jax.experimental.pallas import tpu as pltpu

_NUM_LEVELS = 8
_TABLE_SIZE = 2 ** 19
_FEAT = 4
_MAX_RES = 2 ** 12
_MIN_RES = 16
_OMEGA = 30.0
_PRIMES = np.array([1, 2654435761, 805459861], dtype=np.uint32)
_OFFSETS = np.array([[(i >> d) & 1 for d in range(3)] for i in range(8)])


def _hash_encode_xla(points, table):
    growth = np.exp((np.log(_MAX_RES) - np.log(_MIN_RES)) / (_NUM_LEVELS - 1))
    scales = jnp.asarray(_MIN_RES * growth ** np.arange(_NUM_LEVELS), dtype=points.dtype)
    scaled = points[None, :, :] * scales[:, None, None]
    floor = jnp.floor(scaled)
    frac = scaled - floor
    base = floor.astype(jnp.uint32)
    offs_u = jnp.asarray(_OFFSETS, dtype=jnp.uint32)
    corner = base[:, :, None, :] + offs_u[None, None, :, :]
    h = corner * jnp.asarray(_PRIMES)
    idx = (h[..., 0] ^ h[..., 1] ^ h[..., 2]) & jnp.uint32(_TABLE_SIZE - 1)
    feats = jax.vmap(lambda t, i: t[i])(table, idx)
    offs_f = jnp.asarray(_OFFSETS, dtype=points.dtype)
    w = jnp.prod(jnp.where(offs_f[None, None] == 1,
                           frac[:, :, None, :], 1.0 - frac[:, :, None, :]), axis=-1)
    interp = jnp.einsum('lnc,lncf->lnf', w, feats)
    n = points.shape[0]
    return jnp.transpose(interp, (1, 0, 2)).reshape(n, _NUM_LEVELS * _FEAT)


def _mlp_kernel(enc_ref, *refs):
    w1 = refs[0:6]
    b1 = refs[6:12]
    w2 = refs[12:16]
    b2 = refs[16:20]
    scalar_ref, density_ref = refs[20], refs[21]

    enc = enc_ref[...]
    h = enc
    for li in range(5):
        h = jnp.sin(jnp.dot(h, w1[li][...], preferred_element_type=jnp.float32, precision=jax.lax.Precision.HIGHEST)
                    + b1[li][...])
    x16 = jnp.dot(h, w1[5][...], preferred_element_type=jnp.float32, precision=jax.lax.Precision.HIGHEST) + b1[5][...]
    density_ref[...] = jnp.maximum(x16[:, :1], 0.0)

    g = jnp.concatenate([x16[:, 1:], enc], axis=-1)
    for li in range(3):
        g = jnp.sin(jnp.dot(g, w2[li][...], preferred_element_type=jnp.float32, precision=jax.lax.Precision.HIGHEST)
                    + b2[li][...])
    scalar_ref[...] = jnp.dot(g, w2[3][...], preferred_element_type=jnp.float32, precision=jax.lax.Precision.HIGHEST) + b2[3][...]


def _fused_mlps(enc, params1, params2):
    n = enc.shape[0]
    B = 1024
    g_total = n // B
    g2 = g_total // 2

    def prep(params, n_sin):
        ws, bs = [], []
        for li, (w, b) in enumerate(zip(params['ws'], params['bs'])):
            s = _OMEGA if li < n_sin else 1.0
            ws.append(w * s)
            bs.append((b * s).reshape(1, -1))
        return ws, bs

    ws1, bs1 = prep(params1, 5)
    ws2, bs2 = prep(params2, 3)

    def whole(a):
        return pl.BlockSpec(a.shape, lambda i, j: (0,) * a.ndim)

    in_specs = [pl.BlockSpec((B, enc.shape[1]), lambda i, j: (i * g2 + j, 0))]
    in_specs += [whole(a) for a in (*ws1, *bs1, *ws2, *bs2)]

    out_specs = [pl.BlockSpec((B, 1), lambda i, j: (i * g2 + j, 0)),
                 pl.BlockSpec((B, 1), lambda i, j: (i * g2 + j, 0))]
    out_shape = [jax.ShapeDtypeStruct((n, 1), jnp.float32),
                 jax.ShapeDtypeStruct((n, 1), jnp.float32)]

    scalar, density = pl.pallas_call(
        _mlp_kernel,
        grid=(2, g2),
        in_specs=in_specs,
        out_specs=out_specs,
        out_shape=out_shape,
        compiler_params=pltpu.CompilerParams(
            dimension_semantics=("parallel", "arbitrary"),
            vmem_limit_bytes=100 * 1024 * 1024,
        ),
    )(enc, *ws1, *bs1, *ws2, *bs2)
    return scalar, jnp.squeeze(density, -1)


def kernel(input_points, table, params1, params2):
    enc = _hash_encode_xla(input_points, table)
    return _fused_mlps(enc, params1, params2)
